# Initial kernel scaffold; baseline (speedup 1.0000x reference)
import jax
import jax.numpy as jnp
from jax.experimental import pallas as pl


def kernel(feature, pos, params):
    raise NotImplementedError("write your pallas kernel here")


# speedup vs baseline: 10.7175x; 10.7175x over previous
import functools

import jax
import jax.numpy as jnp
from jax import lax
from jax.experimental import pallas as pl
from jax.experimental.pallas import tpu as pltpu
from jax.experimental.pallas import tpu_sc as plsc


def _knn_body(xt_ref, x_ref, out_ref, *, k, n):
    b = pl.program_id(0)
    xtb = xt_ref[0]
    xb = x_ref[0]
    inner = lax.dot_general(xtb, xb, (((1,), (0,)), ((), ())),
                            preferred_element_type=jnp.float32)
    xx_row = jnp.sum(xb * xb, axis=0, keepdims=True)
    xx_col = jnp.sum(xtb * xtb, axis=1, keepdims=True)
    dist = (xx_col - 2.0 * inner) + xx_row
    lane = lax.broadcasted_iota(jnp.int32, dist.shape, 1)
    base = b * n
    big = jnp.int32(0x7FFFFFFF)
    cols = []
    for j in range(k):
        m = jnp.min(dist, axis=1, keepdims=True)
        hit = dist == m
        idxj = jnp.min(jnp.where(hit, lane, big), axis=1, keepdims=True)
        cols.append(idxj + base)
        if j + 1 < k:
            dist = jnp.where(lane == idxj, jnp.float32(jnp.inf), dist)
    out_ref[0] = jnp.concatenate(cols, axis=1)


def _knn(xr, k):
    B, N, C = xr.shape
    x = xr.transpose(0, 2, 1)
    R = 256
    return pl.pallas_call(
        functools.partial(_knn_body, k=k, n=N),
        grid=(B, N // R),
        in_specs=[pl.BlockSpec((1, R, C), lambda b, r: (b, r, 0)),
                  pl.BlockSpec((1, C, N), lambda b, r: (b, 0, 0))],
        out_specs=pl.BlockSpec((1, R, k), lambda b, r: (b, r, 0)),
        out_shape=jax.ShapeDtypeStruct((B, N, k), jnp.int32),
    )(xr, x)


def _mm_body(x_ref, w_ref, b_ref, o_ref, *, act):
    h = jnp.dot(x_ref[...], w_ref[...],
                preferred_element_type=jnp.float32) + b_ref[...]
    o_ref[...] = jnp.maximum(h, 0.0) if act else h


def _mm(x, w, b, act, R=1024):
    M, Cin = x.shape
    Cout = w.shape[1]
    return pl.pallas_call(
        functools.partial(_mm_body, act=act),
        grid=(M // R,),
        in_specs=[pl.BlockSpec((R, Cin), lambda i: (i, 0)),
                  pl.BlockSpec((Cin, Cout), lambda i: (0, 0)),
                  pl.BlockSpec((1, Cout), lambda i: (0, 0))],
        out_specs=pl.BlockSpec((R, Cout), lambda i: (i, 0)),
        out_shape=jax.ShapeDtypeStruct((M, Cout), jnp.float32),
    )(x, w, b.reshape(1, -1))


def _dual_mm_body(x_ref, wa_ref, wt_ref, b_ref, ya_ref, t_ref):
    xb = x_ref[...]
    ya_ref[...] = jnp.dot(xb, wa_ref[...], preferred_element_type=jnp.float32)
    t_ref[...] = jnp.dot(xb, wt_ref[...],
                         preferred_element_type=jnp.float32) + b_ref[...]


def _dual_mm(x, wa, wt, b, R=1024):
    M, Cin = x.shape
    Cout = wa.shape[1]
    return pl.pallas_call(
        _dual_mm_body,
        grid=(M // R,),
        in_specs=[pl.BlockSpec((R, Cin), lambda i: (i, 0)),
                  pl.BlockSpec((Cin, Cout), lambda i: (0, 0)),
                  pl.BlockSpec((Cin, Cout), lambda i: (0, 0)),
                  pl.BlockSpec((1, Cout), lambda i: (0, 0))],
        out_specs=[pl.BlockSpec((R, Cout), lambda i: (i, 0)),
                   pl.BlockSpec((R, Cout), lambda i: (i, 0))],
        out_shape=[jax.ShapeDtypeStruct((M, Cout), jnp.float32),
                   jax.ShapeDtypeStruct((M, Cout), jnp.float32)],
    )(x, wa, wt, b.reshape(1, -1))


def _edge2_body(g_ref, t_ref, w2_ref, b2_ref, o_ref, *, k):
    R, C1 = t_ref.shape
    t = t_ref[...].reshape(R, 1, C1)
    g = g_ref[...].reshape(R, k, C1)
    h1 = jnp.maximum(g + t, 0.0).reshape(R * k, C1)
    h2 = jnp.maximum(jnp.dot(h1, w2_ref[...],
                             preferred_element_type=jnp.float32)
                     + b2_ref[...], 0.0)
    C2 = h2.shape[1]
    o_ref[...] = jnp.max(h2.reshape(R, k, C2), axis=1)


def _edge2(g, t, w2, b2, k, R=128):
    M, C1 = t.shape
    C2 = w2.shape[1]
    return pl.pallas_call(
        functools.partial(_edge2_body, k=k),
        grid=(M // R,),
        in_specs=[pl.BlockSpec((R * k, C1), lambda i: (i, 0)),
                  pl.BlockSpec((R, C1), lambda i: (i, 0)),
                  pl.BlockSpec((C1, C2), lambda i: (0, 0)),
                  pl.BlockSpec((1, C2), lambda i: (0, 0))],
        out_specs=pl.BlockSpec((R, C2), lambda i: (i, 0)),
        out_shape=jax.ShapeDtypeStruct((M, C2), jnp.float32),
    )(g, t, w2, b2.reshape(1, -1))


def _edge1_max_body(g_ref, t_ref, res_ref, o_ref, *, k):
    R, C1 = t_ref.shape
    gm = jnp.max(g_ref[...].reshape(R, k, C1), axis=1)
    o_ref[...] = jnp.maximum(gm + t_ref[...], 0.0) + res_ref[...]


def _edge1_max(g, t, res, k, R=256):
    M, C1 = t.shape
    return pl.pallas_call(
        functools.partial(_edge1_max_body, k=k),
        grid=(M // R,),
        in_specs=[pl.BlockSpec((R * k, C1), lambda i: (i, 0)),
                  pl.BlockSpec((R, C1), lambda i: (i, 0)),
                  pl.BlockSpec((R, C1), lambda i: (i, 0))],
        out_specs=pl.BlockSpec((R, C1), lambda i: (i, 0)),
        out_shape=jax.ShapeDtypeStruct((M, C1), jnp.float32),
    )(g, t, res)


def _edge1_sum_body(g_ref, t_ref, o_ref, *, k):
    R, C1 = t_ref.shape
    t = t_ref[...].reshape(R, 1, C1)
    h = jnp.maximum(g_ref[...].reshape(R, k, C1) + t, 0.0)
    o_ref[...] = jnp.sum(h, axis=1)


def _edge1_sum(g, t, k, R=256):
    M, C1 = t.shape
    return pl.pallas_call(
        functools.partial(_edge1_sum_body, k=k),
        grid=(M // R,),
        in_specs=[pl.BlockSpec((R * k, C1), lambda i: (i, 0)),
                  pl.BlockSpec((R, C1), lambda i: (i, 0))],
        out_specs=pl.BlockSpec((R, C1), lambda i: (i, 0)),
        out_shape=jax.ShapeDtypeStruct((M, C1), jnp.float32),
    )(g, t)


def _sc_gather(table, idx):
    M = idx.shape[0]
    T, D = table.shape
    NW = 32
    per_w = M // NW
    CH = 128
    nch = per_w // CH
    mesh = plsc.VectorSubcoreMesh(core_axis_name="c", subcore_axis_name="s")

    @functools.partial(
        pl.kernel,
        out_type=jax.ShapeDtypeStruct((M, D), jnp.float32),
        mesh=mesh,
        scratch_types=[pltpu.VMEM((CH,), jnp.int32),
                       pltpu.VMEM((CH, D), jnp.float32),
                       pltpu.SemaphoreType.DMA],
    )
    def kfn(table_hbm, idx_hbm, out_hbm, idx_v, rows_v, sem):
        wid = lax.axis_index("s") * 2 + lax.axis_index("c")
        base = wid * per_w

        def body(i, carry):
            off = base + i * CH
            pltpu.sync_copy(idx_hbm.at[pl.ds(off, CH)], idx_v)
            pltpu.async_copy(table_hbm.at[idx_v], rows_v, sem).wait()
            pltpu.sync_copy(rows_v, out_hbm.at[pl.ds(off, CH)])
            return carry

        lax.fori_loop(0, nch, body, 0)

    return kfn(table, idx)


def _final_body(m_ref, w3_ref, b3_ref, edge_ref, pos_ref, o_ref, mask_ref):
    g = (jnp.sum(m_ref[...] * w3_ref[...], axis=1, keepdims=True)
         + b3_ref[...])
    mask_ref[...] = jnp.maximum(g, 0.0)
    keep = (g > 0.01).astype(jnp.float32)
    o_ref[...] = pos_ref[...] + edge_ref[...] * keep


def _final(m64, w3, b3, edge24, pos24, R=1024):
    M = m64.shape[0]
    return pl.pallas_call(
        _final_body,
        grid=(M // R,),
        in_specs=[pl.BlockSpec((R, 64), lambda i: (i, 0)),
                  pl.BlockSpec((1, 64), lambda i: (0, 0)),
                  pl.BlockSpec((1, 1), lambda i: (0, 0)),
                  pl.BlockSpec((R, 24), lambda i: (i, 0)),
                  pl.BlockSpec((R, 24), lambda i: (i, 0))],
        out_specs=[pl.BlockSpec((R, 24), lambda i: (i, 0)),
                   pl.BlockSpec((R, 1), lambda i: (i, 0))],
        out_shape=[jax.ShapeDtypeStruct((M, 24), jnp.float32),
                   jax.ShapeDtypeStruct((M, 1), jnp.float32)],
    )(m64, w3.reshape(1, 64), b3.reshape(1, 1), edge24, pos24)


def _split_w(w, c):
    wa = w[:, :c]
    return wa.T, (w[:, c:] - wa).T


def kernel(feature, pos, params):
    p = params
    B, N, _ = feature.shape
    BN = B * N

    f0 = feature.reshape(BN, 3)
    x8 = jnp.pad(f0, ((0, 0), (0, 5)))
    wa, wt = _split_w(p['fe0_w1'], 3)
    wa8 = jnp.pad(wa, ((0, 5), (0, 0)))
    wt8 = jnp.pad(wt, ((0, 5), (0, 0)))
    idx = _knn(x8.reshape(B, N, 8), 20)
    ya, t = _dual_mm(x8, wa8, wt8, p['fe0_b1'])
    g = _sc_gather(ya, idx.reshape(-1))
    x1 = _edge2(g, t, p['fe0_w2'].T, p['fe0_b2'], k=20)

    idx = _knn(x1.reshape(B, N, 128), 20)
    wa, wt = _split_w(p['fe1_w'], 128)
    ya, t = _dual_mm(x1, wa, wt, p['fe1_b'])
    g = _sc_gather(ya, idx.reshape(-1))
    f1 = _edge1_max(g, t, x1, k=20)

    idx = _knn(f1.reshape(B, N, 128), 20)
    wa, wt = _split_w(p['fe2_w'], 128)
    ya, t = _dual_mm(f1, wa, wt, p['fe2_b'])
    g = _sc_gather(ya, idx.reshape(-1))
    f2 = _edge1_max(g, t, f1, k=20)

    enc = jnp.concatenate([f1, f2], axis=1)

    h = _mm(enc, p['up0_w'].T, p['up0_b'], act=True)
    idx = _knn(h.reshape(B, N, 64), 12)
    wa, wt = _split_w(p['up1_w1'], 64)
    ya, t = _dual_mm(h, wa, wt, p['up1_b1'])
    g = _sc_gather(ya, idx.reshape(-1))
    h = _edge2(g, t, p['up1_w2'].T, p['up1_b2'], k=12)
    h = _mm(h, p['up2_w'].T, p['up2_b'], act=True)
    idx = _knn(h.reshape(B, N, 64), 4)
    wa, wt = _split_w(p['up3_w1'], 64)
    ya, t = _dual_mm(h, wa, wt, p['up3_b1'])
    g = _sc_gather(ya, idx.reshape(-1))
    h = _edge2(g, t, p['up3_w2'].T, p['up3_b2'], k=4)
    h = _mm(h, p['updec_w1'].T, p['updec_b1'], act=True)
    h = _mm(h, p['updec_w2'].T, p['updec_b2'], act=True)
    edge24 = _mm(h, p['updec_w3'].T, p['updec_b3'], act=False)

    gb = _mm(enc, p['fb0_w'].T, p['fb0_b'], act=True)
    idx = _knn(gb.reshape(B, N, 64), 12)
    wa, wt = _split_w(p['fb1_w1'], 64)
    ya, t = _dual_mm(gb, wa, wt, p['fb1_b1'])
    g = _sc_gather(ya, idx.reshape(-1))
    gb = _edge2(g, t, p['fb1_w2'].T, p['fb1_b2'], k=12)
    gb = _mm(gb, p['fb2_w'].T, p['fb2_b'], act=True)
    idx = _knn(gb.reshape(B, N, 64), 8)
    wa, wt = _split_w(p['fb3_w'], 64)
    ya, t = _dual_mm(gb, wa, wt, p['fb3_b'])
    g = _sc_gather(ya, idx.reshape(-1))
    gb = _edge1_sum(g, t, k=8)
    gb = _mm(gb, p['fbdec_w1'].T, p['fbdec_b1'], act=True)
    m64 = _mm(gb, p['fbdec_w2'].T, p['fbdec_b2'], act=True)

    pos24 = jnp.concatenate([pos] * 8, axis=2).reshape(BN, 24)
    out24, mask = _final(m64, p['fbdec_w3'], p['fbdec_b3'], edge24, pos24)
    return out24.reshape(B, N * 8, 3), mask.reshape(B, N, 1)


# speedup vs baseline: 11.0494x; 1.0310x over previous
import functools

import jax
import jax.numpy as jnp
from jax import lax
from jax.experimental import pallas as pl
from jax.experimental.pallas import tpu as pltpu
from jax.experimental.pallas import tpu_sc as plsc


def _knn_body(xt_ref, x_ref, out_ref, *, k, n):
    b = pl.program_id(0)
    xtb = xt_ref[0]
    xb = x_ref[0]
    inner = lax.dot_general(xtb, xb, (((1,), (0,)), ((), ())),
                            preferred_element_type=jnp.float32)
    xx_row = jnp.sum(xb * xb, axis=0, keepdims=True)
    xx_col = jnp.sum(xtb * xtb, axis=1, keepdims=True)
    dist = (xx_col - 2.0 * inner) + xx_row
    lane = lax.broadcasted_iota(jnp.int32, dist.shape, 1)
    base = b * n
    big = jnp.int32(0x7FFFFFFF)
    cols = []
    for j in range(k):
        m = jnp.min(dist, axis=1, keepdims=True)
        hit = dist == m
        idxj = jnp.min(jnp.where(hit, lane, big), axis=1, keepdims=True)
        cols.append(idxj + base)
        if j + 1 < k:
            dist = jnp.where(lane == idxj, jnp.float32(jnp.inf), dist)
    out_ref[0] = jnp.concatenate(cols, axis=1)


def _knn(xr, k):
    B, N, C = xr.shape
    x = xr.transpose(0, 2, 1)
    R = 256
    return pl.pallas_call(
        functools.partial(_knn_body, k=k, n=N),
        grid=(B, N // R),
        in_specs=[pl.BlockSpec((1, R, C), lambda b, r: (b, r, 0)),
                  pl.BlockSpec((1, C, N), lambda b, r: (b, 0, 0))],
        out_specs=pl.BlockSpec((1, R, k), lambda b, r: (b, r, 0)),
        out_shape=jax.ShapeDtypeStruct((B, N, k), jnp.int32),
    )(xr, x)


def _mm_body(x_ref, w_ref, b_ref, o_ref, *, act):
    h = jnp.dot(x_ref[...], w_ref[...],
                preferred_element_type=jnp.float32) + b_ref[...]
    o_ref[...] = jnp.maximum(h, 0.0) if act else h


def _mm(x, w, b, act, R=1024):
    M, Cin = x.shape
    Cout = w.shape[1]
    return pl.pallas_call(
        functools.partial(_mm_body, act=act),
        grid=(M // R,),
        in_specs=[pl.BlockSpec((R, Cin), lambda i: (i, 0)),
                  pl.BlockSpec((Cin, Cout), lambda i: (0, 0)),
                  pl.BlockSpec((1, Cout), lambda i: (0, 0))],
        out_specs=pl.BlockSpec((R, Cout), lambda i: (i, 0)),
        out_shape=jax.ShapeDtypeStruct((M, Cout), jnp.float32),
    )(x, w, b.reshape(1, -1))


def _dual_mm_body(x_ref, wa_ref, wt_ref, b_ref, ya_ref, t_ref):
    xb = x_ref[...]
    ya_ref[...] = jnp.dot(xb, wa_ref[...], preferred_element_type=jnp.float32)
    t_ref[...] = jnp.dot(xb, wt_ref[...],
                         preferred_element_type=jnp.float32) + b_ref[...]


def _dual_mm(x, wa, wt, b, R=1024):
    M, Cin = x.shape
    Cout = wa.shape[1]
    return pl.pallas_call(
        _dual_mm_body,
        grid=(M // R,),
        in_specs=[pl.BlockSpec((R, Cin), lambda i: (i, 0)),
                  pl.BlockSpec((Cin, Cout), lambda i: (0, 0)),
                  pl.BlockSpec((Cin, Cout), lambda i: (0, 0)),
                  pl.BlockSpec((1, Cout), lambda i: (0, 0))],
        out_specs=[pl.BlockSpec((R, Cout), lambda i: (i, 0)),
                   pl.BlockSpec((R, Cout), lambda i: (i, 0))],
        out_shape=[jax.ShapeDtypeStruct((M, Cout), jnp.float32),
                   jax.ShapeDtypeStruct((M, Cout), jnp.float32)],
    )(x, wa, wt, b.reshape(1, -1))


def _edge2_body(g_ref, t_ref, wa_ref, w2_ref, b2_ref, o_ref, *, k):
    R, C1 = t_ref.shape
    t = t_ref[...].reshape(R, 1, C1)
    if wa_ref is not None:
        ga = jnp.dot(g_ref[...], wa_ref[...],
                     preferred_element_type=jnp.float32)
    else:
        ga = g_ref[...]
    h1 = jnp.maximum(ga.reshape(R, k, C1) + t, 0.0).reshape(R * k, C1)
    h2 = jnp.maximum(jnp.dot(h1, w2_ref[...],
                             preferred_element_type=jnp.float32)
                     + b2_ref[...], 0.0)
    C2 = h2.shape[1]
    o_ref[...] = jnp.max(h2.reshape(R, k, C2), axis=1)


def _edge2(g, t, wa, w2, b2, k, R=128):
    M, C1 = t.shape
    C = g.shape[1]
    C2 = w2.shape[1]
    specs = [pl.BlockSpec((R * k, C), lambda i: (i, 0)),
             pl.BlockSpec((R, C1), lambda i: (i, 0))]
    args = [g, t]
    if wa is not None:
        specs.append(pl.BlockSpec((C, C1), lambda i: (0, 0)))
        args.append(wa)
        body = functools.partial(_edge2_body, k=k)
    else:
        body = functools.partial(
            lambda g_r, t_r, w2_r, b2_r, o_r, k: _edge2_body(
                g_r, t_r, None, w2_r, b2_r, o_r, k=k), k=k)
    specs += [pl.BlockSpec((C1, C2), lambda i: (0, 0)),
              pl.BlockSpec((1, C2), lambda i: (0, 0))]
    args += [w2, b2.reshape(1, -1)]
    return pl.pallas_call(
        body,
        grid=(M // R,),
        in_specs=specs,
        out_specs=pl.BlockSpec((R, C2), lambda i: (i, 0)),
        out_shape=jax.ShapeDtypeStruct((M, C2), jnp.float32),
    )(*args)


def _edge1_max_body(g_ref, t_ref, res_ref, o_ref, *, k):
    R, C1 = t_ref.shape
    gm = jnp.max(g_ref[...].reshape(R, k, C1), axis=1)
    o_ref[...] = jnp.maximum(gm + t_ref[...], 0.0) + res_ref[...]


def _edge1_max(g, t, res, k, R=256):
    M, C1 = t.shape
    return pl.pallas_call(
        functools.partial(_edge1_max_body, k=k),
        grid=(M // R,),
        in_specs=[pl.BlockSpec((R * k, C1), lambda i: (i, 0)),
                  pl.BlockSpec((R, C1), lambda i: (i, 0)),
                  pl.BlockSpec((R, C1), lambda i: (i, 0))],
        out_specs=pl.BlockSpec((R, C1), lambda i: (i, 0)),
        out_shape=jax.ShapeDtypeStruct((M, C1), jnp.float32),
    )(g, t, res)


def _edge1_sum_body(g_ref, t_ref, wa_ref, o_ref, *, k):
    R, C1 = t_ref.shape
    t = t_ref[...].reshape(R, 1, C1)
    ga = jnp.dot(g_ref[...], wa_ref[...], preferred_element_type=jnp.float32)
    h = jnp.maximum(ga.reshape(R, k, C1) + t, 0.0)
    o_ref[...] = jnp.sum(h, axis=1)


def _edge1_sum(g, t, wa, k, R=256):
    M, C1 = t.shape
    C = g.shape[1]
    return pl.pallas_call(
        functools.partial(_edge1_sum_body, k=k),
        grid=(M // R,),
        in_specs=[pl.BlockSpec((R * k, C), lambda i: (i, 0)),
                  pl.BlockSpec((R, C1), lambda i: (i, 0)),
                  pl.BlockSpec((C, C1), lambda i: (0, 0))],
        out_specs=pl.BlockSpec((R, C1), lambda i: (i, 0)),
        out_shape=jax.ShapeDtypeStruct((M, C1), jnp.float32),
    )(g, t, wa)


def _sc_gather(table, idx):
    M = idx.shape[0]
    T, D = table.shape
    NW = 32
    per_w = M // NW
    CH = 128
    nch = per_w // CH
    mesh = plsc.VectorSubcoreMesh(core_axis_name="c", subcore_axis_name="s")

    @functools.partial(
        pl.kernel,
        out_type=jax.ShapeDtypeStruct((M, D), jnp.float32),
        mesh=mesh,
        scratch_types=[pltpu.VMEM((2, CH), jnp.int32),
                       pltpu.VMEM((2, CH, D), jnp.float32),
                       pltpu.SemaphoreType.DMA],
    )
    def kfn(table_hbm, idx_hbm, out_hbm, idx_v, rows_v, gsem):
        wid = lax.axis_index("s") * 2 + lax.axis_index("c")
        base = wid * per_w
        pltpu.sync_copy(idx_hbm.at[pl.ds(base, CH)], idx_v.at[0])
        pltpu.async_copy(table_hbm.at[idx_v.at[0]], rows_v.at[0], gsem)

        def body(i, carry):
            cur = lax.rem(i, 2)
            nxt = 1 - cur
            pltpu.make_async_copy(table_hbm.at[idx_v.at[cur]],
                                  rows_v.at[cur], gsem).wait()

            @pl.when(i + 1 < nch)
            def _prefetch():
                off = base + (i + 1) * CH
                pltpu.sync_copy(idx_hbm.at[pl.ds(off, CH)], idx_v.at[nxt])
                pltpu.async_copy(table_hbm.at[idx_v.at[nxt]],
                                 rows_v.at[nxt], gsem)

            pltpu.sync_copy(rows_v.at[cur],
                            out_hbm.at[pl.ds(base + i * CH, CH)])
            return carry

        lax.fori_loop(0, nch, body, 0)

    return kfn(table, idx)


def _final_body(m_ref, w3_ref, b3_ref, edge_ref, pos_ref, o_ref, mask_ref):
    g = (jnp.sum(m_ref[...] * w3_ref[...], axis=1, keepdims=True)
         + b3_ref[...])
    mask_ref[...] = jnp.maximum(g, 0.0)
    keep = (g > 0.01).astype(jnp.float32)
    o_ref[...] = pos_ref[...] + edge_ref[...] * keep


def _final(m64, w3, b3, edge24, pos24, R=1024):
    M = m64.shape[0]
    return pl.pallas_call(
        _final_body,
        grid=(M // R,),
        in_specs=[pl.BlockSpec((R, 64), lambda i: (i, 0)),
                  pl.BlockSpec((1, 64), lambda i: (0, 0)),
                  pl.BlockSpec((1, 1), lambda i: (0, 0)),
                  pl.BlockSpec((R, 24), lambda i: (i, 0)),
                  pl.BlockSpec((R, 24), lambda i: (i, 0))],
        out_specs=[pl.BlockSpec((R, 24), lambda i: (i, 0)),
                   pl.BlockSpec((R, 1), lambda i: (i, 0))],
        out_shape=[jax.ShapeDtypeStruct((M, 24), jnp.float32),
                   jax.ShapeDtypeStruct((M, 1), jnp.float32)],
    )(m64, w3.reshape(1, 64), b3.reshape(1, 1), edge24, pos24)


def _split_w(w, c, cp=None):
    wa = w[:, :c]
    waT, wtT = wa.T, (w[:, c:] - wa).T
    if cp is not None:
        waT = jnp.pad(waT, ((0, cp - c), (0, 0)))
        wtT = jnp.pad(wtT, ((0, cp - c), (0, 0)))
    return waT, wtT


def _pad128(wT, b):
    co = wT.shape[1]
    return jnp.pad(wT, ((0, 0), (0, 128 - co))), jnp.pad(b, (0, 128 - co))


def kernel(feature, pos, params):
    p = params
    B, N, _ = feature.shape
    BN = B * N

    f0 = feature.reshape(BN, 3)
    x16 = jnp.pad(f0, ((0, 0), (0, 13)))
    wa, wt = _split_w(p['fe0_w1'], 3)
    wa16 = jnp.pad(wa, ((0, 13), (0, 0)))
    wt16 = jnp.pad(wt, ((0, 13), (0, 0)))
    idx = _knn(x16.reshape(B, N, 16), 20)
    ya, t = _dual_mm(x16, wa16, wt16, p['fe0_b1'])
    g = _sc_gather(ya, idx.reshape(-1))
    x1 = _edge2(g, t, None, p['fe0_w2'].T, p['fe0_b2'], k=20)

    idx = _knn(x1.reshape(B, N, 128), 20)
    wa, wt = _split_w(p['fe1_w'], 128)
    ya, t = _dual_mm(x1, wa, wt, p['fe1_b'])
    g = _sc_gather(ya, idx.reshape(-1))
    f1 = _edge1_max(g, t, x1, k=20)

    idx = _knn(f1.reshape(B, N, 128), 20)
    wa, wt = _split_w(p['fe2_w'], 128)
    ya, t = _dual_mm(f1, wa, wt, p['fe2_b'])
    g = _sc_gather(ya, idx.reshape(-1))
    f2 = _edge1_max(g, t, f1, k=20)

    enc = jnp.concatenate([f1, f2], axis=1)

    w0, b0 = _pad128(p['up0_w'].T, p['up0_b'])
    h = _mm(enc, w0, b0, act=True)
    idx = _knn(h.reshape(B, N, 128), 12)
    wa, wt = _split_w(p['up1_w1'], 64, 128)
    t = _mm(h, wt, p['up1_b1'], act=False)
    g = _sc_gather(h, idx.reshape(-1))
    h = _edge2(g, t, wa, p['up1_w2'].T, p['up1_b2'], k=12)
    w0, b0 = _pad128(p['up2_w'].T, p['up2_b'])
    h = _mm(h, w0, b0, act=True)
    idx = _knn(h.reshape(B, N, 128), 4)
    wa, wt = _split_w(p['up3_w1'], 64, 128)
    t = _mm(h, wt, p['up3_b1'], act=False)
    g = _sc_gather(h, idx.reshape(-1))
    h = _edge2(g, t, wa, p['up3_w2'].T, p['up3_b2'], k=4)
    h = _mm(h, p['updec_w1'].T, p['updec_b1'], act=True)
    h = _mm(h, p['updec_w2'].T, p['updec_b2'], act=True)
    edge24 = _mm(h, p['updec_w3'].T, p['updec_b3'], act=False)

    w0, b0 = _pad128(p['fb0_w'].T, p['fb0_b'])
    gb = _mm(enc, w0, b0, act=True)
    idx = _knn(gb.reshape(B, N, 128), 12)
    wa, wt = _split_w(p['fb1_w1'], 64, 128)
    t = _mm(gb, wt, p['fb1_b1'], act=False)
    g = _sc_gather(gb, idx.reshape(-1))
    gb = _edge2(g, t, wa, p['fb1_w2'].T, p['fb1_b2'], k=12)
    w0, b0 = _pad128(p['fb2_w'].T, p['fb2_b'])
    gb = _mm(gb, w0, b0, act=True)
    idx = _knn(gb.reshape(B, N, 128), 8)
    wa, wt = _split_w(p['fb3_w'], 64, 128)
    t = _mm(gb, wt, p['fb3_b'], act=False)
    g = _sc_gather(gb, idx.reshape(-1))
    gb = _edge1_sum(g, t, wa, k=8)
    gb = _mm(gb, p['fbdec_w1'].T, p['fbdec_b1'], act=True)
    m64 = _mm(gb, p['fbdec_w2'].T, p['fbdec_b2'], act=True)

    pos24 = jnp.concatenate([pos] * 8, axis=2).reshape(BN, 24)
    out24, mask = _final(m64, p['fbdec_w3'], p['fbdec_b3'], edge24, pos24)
    return out24.reshape(B, N * 8, 3), mask.reshape(B, N, 1)
